# Initial kernel scaffold; baseline (speedup 1.0000x reference)
import jax
import jax.numpy as jnp
from jax.experimental import pallas as pl


def kernel(x, edge_index, batch, W1, b1, W2, b2, W3, b3, Wfc, bfc):
    raise NotImplementedError("write your pallas kernel here")


# speedup vs baseline: 9.0796x; 9.0796x over previous
import functools

import jax
import jax.numpy as jnp
from jax import lax
from jax.experimental import pallas as pl
from jax.experimental.pallas import tpu as pltpu
from jax.experimental.pallas import tpu_sc as plsc

_N = 10000
_E = 320000
_D = 128
_B = 64
_C = 16

_K = 128
_NSC = 2
_NTEC = 16
_NW = _NSC * _NTEC
_NPAD = 10240
_STRIPE = _NPAD // _NTEC
_EPAD = ((_E + _NW * _K - 1) // (_NW * _K)) * (_NW * _K)
_PER_W = _EPAD // _NW
_NCHUNK = _PER_W // _K

_BLK = 640
_NBLK = _NPAD // _BLK


@functools.lru_cache(maxsize=None)
def _build_deg_kernel():
    mesh = plsc.VectorSubcoreMesh(core_axis_name="c", subcore_axis_name="s")

    @functools.partial(
        pl.kernel,
        mesh=mesh,
        out_type=jax.ShapeDtypeStruct((_NSC, _NPAD, _D), jnp.float32),
        scratch_types=[
            pltpu.VMEM((_K,), jnp.int32),
            pltpu.VMEM((_K, _D), jnp.float32),
            pltpu.VMEM_SHARED((_NPAD, _D), jnp.float32),
        ],
    )
    def deg_kernel(dst_hbm, out_hbm, didx_v, buf_v, acc_sh):
        c = lax.axis_index("c")
        s = lax.axis_index("s")

        def fill(val):
            def body(i, carry):
                for j in range(_D // 16):
                    buf_v[i, pl.ds(j * 16, 16)] = jnp.full((16,), val,
                                                           jnp.float32)
                return carry
            lax.fori_loop(0, _K, body, 0)

        fill(0.0)
        for blk in range(_STRIPE // _K):
            pltpu.sync_copy(buf_v,
                            acc_sh.at[pl.ds(s * _STRIPE + blk * _K, _K), :])
        fill(1.0)
        plsc.subcore_barrier()

        wid = c * _NTEC + s
        base = wid * _PER_W

        def chunk(j, carry):
            pltpu.sync_copy(dst_hbm.at[pl.ds(base + j * _K, _K)], didx_v)
            pltpu.sync_copy(buf_v, acc_sh.at[didx_v], add=True)
            return carry

        lax.fori_loop(0, _NCHUNK, chunk, 0)
        plsc.subcore_barrier()
        pltpu.sync_copy(
            acc_sh.at[pl.ds(s * _STRIPE, _STRIPE), :],
            out_hbm.at[c, pl.ds(s * _STRIPE, _STRIPE), :],
        )

    return deg_kernel


@functools.lru_cache(maxsize=None)
def _build_edge_kernel():
    mesh = plsc.VectorSubcoreMesh(core_axis_name="c", subcore_axis_name="s")

    @functools.partial(
        pl.kernel,
        mesh=mesh,
        out_type=jax.ShapeDtypeStruct((_NSC, _NPAD, _D), jnp.float32),
        scratch_types=[
            pltpu.VMEM((_K,), jnp.int32),
            pltpu.VMEM((_K,), jnp.int32),
            pltpu.VMEM((_K, _D), jnp.float32),
            pltpu.VMEM_SHARED((_NPAD, _D), jnp.float32),
            pltpu.SemaphoreType.DMA,
        ],
    )
    def edge_kernel(table_hbm, src_hbm, dst_hbm, out_hbm,
                    sidx_v, didx_v, rows_v, acc_sh, sem):
        c = lax.axis_index("c")
        s = lax.axis_index("s")

        def zrow(i, carry):
            for j in range(_D // 16):
                rows_v[i, pl.ds(j * 16, 16)] = jnp.zeros((16,), jnp.float32)
            return carry
        lax.fori_loop(0, _K, zrow, 0)
        for blk in range(_STRIPE // _K):
            pltpu.sync_copy(rows_v,
                            acc_sh.at[pl.ds(s * _STRIPE + blk * _K, _K), :])
        plsc.subcore_barrier()

        wid = c * _NTEC + s
        base = wid * _PER_W

        def chunk(j, carry):
            off = base + j * _K
            pltpu.sync_copy(src_hbm.at[pl.ds(off, _K)], sidx_v)
            pltpu.sync_copy(dst_hbm.at[pl.ds(off, _K)], didx_v)
            pltpu.async_copy(table_hbm.at[sidx_v], rows_v, sem).wait()
            pltpu.sync_copy(rows_v, acc_sh.at[didx_v], add=True)
            return carry

        lax.fori_loop(0, _NCHUNK, chunk, 0)
        plsc.subcore_barrier()
        pltpu.sync_copy(
            acc_sh.at[pl.ds(s * _STRIPE, _STRIPE), :],
            out_hbm.at[c, pl.ds(s * _STRIPE, _STRIPE), :],
        )

    return edge_kernel


def _dinv_block(degp_ref, i):
    deg = degp_ref[0, :, 0:1] + degp_ref[1, :, 0:1] + 1.0
    dinv = lax.rsqrt(jnp.maximum(deg, 1.0))
    row = lax.broadcasted_iota(jnp.int32, (_BLK, 1), 0) + i * _BLK
    dinvm = jnp.where(row < _N, dinv, 0.0)
    return dinv, dinvm


def _t1_body(degp_ref, x_ref, w_ref, out_ref):
    i = pl.program_id(0)
    _, dinvm = _dinv_block(degp_ref, i)
    xw = jnp.dot(x_ref[...], w_ref[...], preferred_element_type=jnp.float32)
    out_ref[...] = xw * dinvm


def _t2_body(degp_ref, p_ref, ys_ref, b_ref, w_ref, out_ref):
    i = pl.program_id(0)
    dinv, dinvm = _dinv_block(degp_ref, i)
    ssum = p_ref[0] + p_ref[1] + ys_ref[...]
    h = jnp.maximum(ssum * dinv + b_ref[...], 0.0)
    out_ref[...] = jnp.dot(h, w_ref[...], preferred_element_type=jnp.float32) * dinvm


def _t3_body(degp_ref, p_ref, ys_ref, b_ref, batch_ref, wfc_ref, bfc_ref,
             out_ref, pooled_acc, cnt_acc):
    i = pl.program_id(0)

    @pl.when(i == 0)
    def _():
        pooled_acc[...] = jnp.zeros_like(pooled_acc)
        cnt_acc[...] = jnp.zeros_like(cnt_acc)

    dinv, _ = _dinv_block(degp_ref, i)
    h3 = (p_ref[0] + p_ref[1] + ys_ref[...]) * dinv + b_ref[...]
    bb = batch_ref[pl.ds(i * _BLK, _BLK)]
    onehot = (bb[None, :] == lax.broadcasted_iota(jnp.int32, (_B, _BLK), 0)
              ).astype(jnp.float32)
    pooled_acc[...] += jnp.dot(onehot, h3, preferred_element_type=jnp.float32)
    cnt_acc[...] += jnp.sum(onehot, axis=1, keepdims=True)

    @pl.when(i == _NBLK - 1)
    def _():
        pooled = pooled_acc[...] / jnp.maximum(cnt_acc[...], 1.0)
        out_ref[...] = (
            jnp.dot(pooled, wfc_ref[...], preferred_element_type=jnp.float32)
            + bfc_ref[...]
        )


def _t1(degp, x_p, W1):
    return pl.pallas_call(
        _t1_body,
        grid=(_NBLK,),
        in_specs=[
            pl.BlockSpec((_NSC, _BLK, _D), lambda i: (0, i, 0)),
            pl.BlockSpec((_BLK, _D), lambda i: (i, 0)),
            pl.BlockSpec((_D, _D), lambda i: (0, 0)),
        ],
        out_specs=pl.BlockSpec((_BLK, _D), lambda i: (i, 0)),
        out_shape=jax.ShapeDtypeStruct((_NPAD, _D), jnp.float32),
    )(degp, x_p, W1)


def _t2(degp, p, ys, b2d, Wn):
    return pl.pallas_call(
        _t2_body,
        grid=(_NBLK,),
        in_specs=[
            pl.BlockSpec((_NSC, _BLK, _D), lambda i: (0, i, 0)),
            pl.BlockSpec((_NSC, _BLK, _D), lambda i: (0, i, 0)),
            pl.BlockSpec((_BLK, _D), lambda i: (i, 0)),
            pl.BlockSpec((1, _D), lambda i: (0, 0)),
            pl.BlockSpec((_D, _D), lambda i: (0, 0)),
        ],
        out_specs=pl.BlockSpec((_BLK, _D), lambda i: (i, 0)),
        out_shape=jax.ShapeDtypeStruct((_NPAD, _D), jnp.float32),
    )(degp, p, ys, b2d, Wn)


def _t3(degp, p, ys, b2d, batch_p, Wfc, bfc2d):
    return pl.pallas_call(
        _t3_body,
        grid=(_NBLK,),
        in_specs=[
            pl.BlockSpec((_NSC, _BLK, _D), lambda i: (0, i, 0)),
            pl.BlockSpec((_NSC, _BLK, _D), lambda i: (0, i, 0)),
            pl.BlockSpec((_BLK, _D), lambda i: (i, 0)),
            pl.BlockSpec((1, _D), lambda i: (0, 0)),
            pl.BlockSpec((_NPAD,), lambda i: (0,)),
            pl.BlockSpec((_D, _C), lambda i: (0, 0)),
            pl.BlockSpec((1, _C), lambda i: (0, 0)),
        ],
        out_specs=pl.BlockSpec((_B, _C), lambda i: (0, 0)),
        out_shape=jax.ShapeDtypeStruct((_B, _C), jnp.float32),
        scratch_shapes=[
            pltpu.VMEM((_B, _D), jnp.float32),
            pltpu.VMEM((_B, 1), jnp.float32),
        ],
    )(degp, p, ys, b2d, batch_p, Wfc, bfc2d)


def kernel(x, edge_index, batch, W1, b1, W2, b2, W3, b3, Wfc, bfc):
    src = edge_index[0]
    dst = edge_index[1]
    pad_e = _EPAD - _E
    fill = jnp.full((pad_e,), _N, jnp.int32)
    src_p = jnp.concatenate([src, fill])
    dst_p = jnp.concatenate([dst, fill])
    x_p = jnp.pad(x, ((0, _NPAD - _N), (0, 0)))
    batch_p = jnp.concatenate(
        [batch, jnp.full((_NPAD - _N,), _B, jnp.int32)])

    edge_k = _build_edge_kernel()
    degp = _build_deg_kernel()(dst_p)
    ys1 = _t1(degp, x_p, W1)
    p1 = edge_k(ys1, src_p, dst_p)
    ys2 = _t2(degp, p1, ys1, b1.reshape(1, _D), W2)
    p2 = edge_k(ys2, src_p, dst_p)
    ys3 = _t2(degp, p2, ys2, b2.reshape(1, _D), W3)
    p3 = edge_k(ys3, src_p, dst_p)
    return _t3(degp, p3, ys3, b3.reshape(1, _D), batch_p, Wfc,
               bfc.reshape(1, _C))


# speedup vs baseline: 11.5918x; 1.2767x over previous
import functools

import jax
import jax.numpy as jnp
from jax import lax
from jax.experimental import pallas as pl
from jax.experimental.pallas import tpu as pltpu
from jax.experimental.pallas import tpu_sc as plsc

_N = 10000
_E = 320000
_D = 128
_B = 64
_C = 16

_K = 128
_NSC = 2
_NTEC = 16
_NW = _NSC * _NTEC
_NPAD = 10240
_STRIPE = _NPAD // _NTEC
_EPAD = ((_E + _NW * _K - 1) // (_NW * _K)) * (_NW * _K)
_PER_W = _EPAD // _NW
_NCHUNK = _PER_W // _K

_BLK = 640
_NBLK = _NPAD // _BLK


@functools.lru_cache(maxsize=None)
def _build_deg_kernel():
    mesh = plsc.VectorSubcoreMesh(core_axis_name="c", subcore_axis_name="s")

    @functools.partial(
        pl.kernel,
        mesh=mesh,
        out_type=jax.ShapeDtypeStruct((_NSC, _NPAD, _D), jnp.float32),
        scratch_types=[
            pltpu.VMEM((_K,), jnp.int32),
            pltpu.VMEM((_K, _D), jnp.float32),
            pltpu.VMEM_SHARED((_NPAD, _D), jnp.float32),
        ],
    )
    def deg_kernel(dst_hbm, out_hbm, didx_v, buf_v, acc_sh):
        c = lax.axis_index("c")
        s = lax.axis_index("s")

        def fill(val):
            def body(i, carry):
                for j in range(_D // 16):
                    buf_v[i, pl.ds(j * 16, 16)] = jnp.full((16,), val,
                                                           jnp.float32)
                return carry
            lax.fori_loop(0, _K, body, 0)

        fill(0.0)
        for blk in range(_STRIPE // _K):
            pltpu.sync_copy(buf_v,
                            acc_sh.at[pl.ds(s * _STRIPE + blk * _K, _K), :])
        fill(1.0)
        plsc.subcore_barrier()

        wid = c * _NTEC + s
        base = wid * _PER_W

        def chunk(j, carry):
            pltpu.sync_copy(dst_hbm.at[pl.ds(base + j * _K, _K)], didx_v)
            pltpu.sync_copy(buf_v, acc_sh.at[didx_v], add=True)
            return carry

        lax.fori_loop(0, _NCHUNK, chunk, 0)
        plsc.subcore_barrier()
        pltpu.sync_copy(
            acc_sh.at[pl.ds(s * _STRIPE, _STRIPE), :],
            out_hbm.at[c, pl.ds(s * _STRIPE, _STRIPE), :],
        )

    return deg_kernel


@functools.lru_cache(maxsize=None)
def _build_edge_kernel():
    mesh = plsc.VectorSubcoreMesh(core_axis_name="c", subcore_axis_name="s")

    @functools.partial(
        pl.kernel,
        mesh=mesh,
        out_type=jax.ShapeDtypeStruct((_NSC, _NPAD, _D), jnp.float32),
        scratch_types=[
            pltpu.VMEM((_K,), jnp.int32),
            pltpu.VMEM((_K,), jnp.int32),
            pltpu.VMEM((_K,), jnp.int32),
            pltpu.VMEM((_K,), jnp.int32),
            pltpu.VMEM((_K, _D), jnp.float32),
            pltpu.VMEM((_K, _D), jnp.float32),
            pltpu.VMEM_SHARED((_NPAD, _D), jnp.float32),
            pltpu.SemaphoreType.DMA,
            pltpu.SemaphoreType.DMA,
        ],
    )
    def edge_kernel(table_hbm, src_hbm, dst_hbm, out_hbm,
                    sidx0, didx0, sidx1, didx1, rows0, rows1, acc_sh,
                    sem0, sem1):
        c = lax.axis_index("c")
        s = lax.axis_index("s")

        def zrow(i, carry):
            for j in range(_D // 16):
                rows0[i, pl.ds(j * 16, 16)] = jnp.zeros((16,), jnp.float32)
            return carry
        lax.fori_loop(0, _K, zrow, 0)
        for blk in range(_STRIPE // _K):
            pltpu.sync_copy(rows0,
                            acc_sh.at[pl.ds(s * _STRIPE + blk * _K, _K), :])
        plsc.subcore_barrier()

        wid = c * _NTEC + s
        base = wid * _PER_W

        def fire(j, sidx, didx, rows, sem):
            off = base + j * _K
            pltpu.sync_copy(src_hbm.at[pl.ds(off, _K)], sidx)
            pltpu.sync_copy(dst_hbm.at[pl.ds(off, _K)], didx)
            pltpu.async_copy(table_hbm.at[sidx], rows, sem)

        def drain(sidx, didx, rows, sem):
            pltpu.make_async_copy(table_hbm.at[sidx], rows, sem).wait()
            pltpu.sync_copy(rows, acc_sh.at[didx], add=True)

        fire(0, sidx0, didx0, rows0, sem0)

        def pair(jj, carry):
            a = 2 * jj
            fire(a + 1, sidx1, didx1, rows1, sem1)
            drain(sidx0, didx0, rows0, sem0)
            fire(a + 2, sidx0, didx0, rows0, sem0)
            drain(sidx1, didx1, rows1, sem1)
            return carry

        lax.fori_loop(0, (_NCHUNK - 1) // 2, pair, 0)
        drain(sidx0, didx0, rows0, sem0)
        plsc.subcore_barrier()
        pltpu.sync_copy(
            acc_sh.at[pl.ds(s * _STRIPE, _STRIPE), :],
            out_hbm.at[c, pl.ds(s * _STRIPE, _STRIPE), :],
        )

    return edge_kernel


def _dinv_block(degp_ref, i):
    deg = degp_ref[0, :, 0:1] + degp_ref[1, :, 0:1] + 1.0
    dinv = lax.rsqrt(jnp.maximum(deg, 1.0))
    row = lax.broadcasted_iota(jnp.int32, (_BLK, 1), 0) + i * _BLK
    dinvm = jnp.where(row < _N, dinv, 0.0)
    return dinv, dinvm


def _t1_body(degp_ref, x_ref, w_ref, out_ref):
    i = pl.program_id(0)
    _, dinvm = _dinv_block(degp_ref, i)
    xw = jnp.dot(x_ref[...], w_ref[...], preferred_element_type=jnp.float32)
    out_ref[...] = xw * dinvm


def _t2_body(degp_ref, p_ref, ys_ref, b_ref, w_ref, out_ref):
    i = pl.program_id(0)
    dinv, dinvm = _dinv_block(degp_ref, i)
    ssum = p_ref[0] + p_ref[1] + ys_ref[...]
    h = jnp.maximum(ssum * dinv + b_ref[...], 0.0)
    out_ref[...] = jnp.dot(h, w_ref[...], preferred_element_type=jnp.float32) * dinvm


def _t3_body(degp_ref, p_ref, ys_ref, b_ref, batch_ref, wfc_ref, bfc_ref,
             out_ref, pooled_acc, cnt_acc):
    i = pl.program_id(0)

    @pl.when(i == 0)
    def _():
        pooled_acc[...] = jnp.zeros_like(pooled_acc)
        cnt_acc[...] = jnp.zeros_like(cnt_acc)

    dinv, _ = _dinv_block(degp_ref, i)
    h3 = (p_ref[0] + p_ref[1] + ys_ref[...]) * dinv + b_ref[...]
    bb = batch_ref[pl.ds(i * _BLK, _BLK)]
    onehot = (bb[None, :] == lax.broadcasted_iota(jnp.int32, (_B, _BLK), 0)
              ).astype(jnp.float32)
    pooled_acc[...] += jnp.dot(onehot, h3, preferred_element_type=jnp.float32)
    cnt_acc[...] += jnp.sum(onehot, axis=1, keepdims=True)

    @pl.when(i == _NBLK - 1)
    def _():
        pooled = pooled_acc[...] / jnp.maximum(cnt_acc[...], 1.0)
        out_ref[...] = (
            jnp.dot(pooled, wfc_ref[...], preferred_element_type=jnp.float32)
            + bfc_ref[...]
        )


def _t1(degp, x_p, W1):
    return pl.pallas_call(
        _t1_body,
        grid=(_NBLK,),
        in_specs=[
            pl.BlockSpec((_NSC, _BLK, _D), lambda i: (0, i, 0)),
            pl.BlockSpec((_BLK, _D), lambda i: (i, 0)),
            pl.BlockSpec((_D, _D), lambda i: (0, 0)),
        ],
        out_specs=pl.BlockSpec((_BLK, _D), lambda i: (i, 0)),
        out_shape=jax.ShapeDtypeStruct((_NPAD, _D), jnp.float32),
    )(degp, x_p, W1)


def _t2(degp, p, ys, b2d, Wn):
    return pl.pallas_call(
        _t2_body,
        grid=(_NBLK,),
        in_specs=[
            pl.BlockSpec((_NSC, _BLK, _D), lambda i: (0, i, 0)),
            pl.BlockSpec((_NSC, _BLK, _D), lambda i: (0, i, 0)),
            pl.BlockSpec((_BLK, _D), lambda i: (i, 0)),
            pl.BlockSpec((1, _D), lambda i: (0, 0)),
            pl.BlockSpec((_D, _D), lambda i: (0, 0)),
        ],
        out_specs=pl.BlockSpec((_BLK, _D), lambda i: (i, 0)),
        out_shape=jax.ShapeDtypeStruct((_NPAD, _D), jnp.float32),
    )(degp, p, ys, b2d, Wn)


def _t3(degp, p, ys, b2d, batch_p, Wfc, bfc2d):
    return pl.pallas_call(
        _t3_body,
        grid=(_NBLK,),
        in_specs=[
            pl.BlockSpec((_NSC, _BLK, _D), lambda i: (0, i, 0)),
            pl.BlockSpec((_NSC, _BLK, _D), lambda i: (0, i, 0)),
            pl.BlockSpec((_BLK, _D), lambda i: (i, 0)),
            pl.BlockSpec((1, _D), lambda i: (0, 0)),
            pl.BlockSpec((_NPAD,), lambda i: (0,)),
            pl.BlockSpec((_D, _C), lambda i: (0, 0)),
            pl.BlockSpec((1, _C), lambda i: (0, 0)),
        ],
        out_specs=pl.BlockSpec((_B, _C), lambda i: (0, 0)),
        out_shape=jax.ShapeDtypeStruct((_B, _C), jnp.float32),
        scratch_shapes=[
            pltpu.VMEM((_B, _D), jnp.float32),
            pltpu.VMEM((_B, 1), jnp.float32),
        ],
    )(degp, p, ys, b2d, batch_p, Wfc, bfc2d)


def kernel(x, edge_index, batch, W1, b1, W2, b2, W3, b3, Wfc, bfc):
    src = edge_index[0]
    dst = edge_index[1]
    pad_e = _EPAD - _E
    fill = jnp.full((pad_e,), _N, jnp.int32)
    src_p = jnp.concatenate([src, fill])
    dst_p = jnp.concatenate([dst, fill])
    x_p = jnp.pad(x, ((0, _NPAD - _N), (0, 0)))
    batch_p = jnp.concatenate(
        [batch, jnp.full((_NPAD - _N,), _B, jnp.int32)])

    edge_k = _build_edge_kernel()
    degp = _build_deg_kernel()(dst_p)
    ys1 = _t1(degp, x_p, W1)
    p1 = edge_k(ys1, src_p, dst_p)
    ys2 = _t2(degp, p1, ys1, b1.reshape(1, _D), W2)
    p2 = edge_k(ys2, src_p, dst_p)
    ys3 = _t2(degp, p2, ys2, b2.reshape(1, _D), W3)
    p3 = edge_k(ys3, src_p, dst_p)
    return _t3(degp, p3, ys3, b3.reshape(1, _D), batch_p, Wfc,
               bfc.reshape(1, _C))


# speedup vs baseline: 13.0561x; 1.1263x over previous
import functools

import jax
import jax.numpy as jnp
from jax import lax
from jax.experimental import pallas as pl
from jax.experimental.pallas import tpu as pltpu
from jax.experimental.pallas import tpu_sc as plsc

_N = 10000
_E = 320000
_D = 128
_B = 64
_C = 16

_K = 128
_NSC = 2
_NTEC = 16
_NW = _NSC * _NTEC
_NPAD = 10240
_STRIPE = _NPAD // _NTEC
_EPAD = ((_E + _NW * _K - 1) // (_NW * _K)) * (_NW * _K)
_PER_W = _EPAD // _NW
_NCHUNK = _PER_W // _K

_NCHUNK_C = (105, 53)

_BLK = 640
_NBLK = _NPAD // _BLK


@functools.lru_cache(maxsize=None)
def _build_deg_kernel():
    mesh = plsc.VectorSubcoreMesh(core_axis_name="c", subcore_axis_name="s")

    @functools.partial(
        pl.kernel,
        mesh=mesh,
        out_type=jax.ShapeDtypeStruct((_NSC, _NPAD, _D), jnp.float32),
        scratch_types=[
            pltpu.VMEM((_K,), jnp.int32),
            pltpu.VMEM((_K, _D), jnp.float32),
            pltpu.VMEM_SHARED((_NPAD, _D), jnp.float32),
        ],
    )
    def deg_kernel(dst_hbm, out_hbm, didx_v, buf_v, acc_sh):
        c = lax.axis_index("c")
        s = lax.axis_index("s")

        def fill(val):
            def body(i, carry):
                for j in range(_D // 16):
                    buf_v[i, pl.ds(j * 16, 16)] = jnp.full((16,), val,
                                                           jnp.float32)
                return carry
            lax.fori_loop(0, _K, body, 0)

        fill(0.0)
        for blk in range(_STRIPE // _K):
            pltpu.sync_copy(buf_v,
                            acc_sh.at[pl.ds(s * _STRIPE + blk * _K, _K), :])
        fill(1.0)
        plsc.subcore_barrier()

        wid = c * _NTEC + s
        base = wid * _PER_W

        def chunk(j, carry):
            pltpu.sync_copy(dst_hbm.at[pl.ds(base + j * _K, _K)], didx_v)
            pltpu.sync_copy(buf_v, acc_sh.at[didx_v], add=True)
            return carry

        lax.fori_loop(0, _NCHUNK, chunk, 0)
        plsc.subcore_barrier()
        pltpu.sync_copy(
            acc_sh.at[pl.ds(s * _STRIPE, _STRIPE), :],
            out_hbm.at[c, pl.ds(s * _STRIPE, _STRIPE), :],
        )

    return deg_kernel


@functools.lru_cache(maxsize=None)
def _build_edge_kernel():
    mesh = plsc.VectorSubcoreMesh(core_axis_name="c", subcore_axis_name="s")

    @functools.partial(
        pl.kernel,
        mesh=mesh,
        out_type=jax.ShapeDtypeStruct((_NSC, _NPAD, _D), jnp.float32),
        scratch_types=[
            pltpu.VMEM((_K,), jnp.int32),
            pltpu.VMEM((_K,), jnp.int32),
            pltpu.VMEM((_K,), jnp.int32),
            pltpu.VMEM((_K,), jnp.int32),
            pltpu.VMEM((_K, _D), jnp.float32),
            pltpu.VMEM((_K, _D), jnp.float32),
            pltpu.VMEM_SHARED((_NPAD, _D), jnp.float32),
            pltpu.SemaphoreType.DMA,
            pltpu.SemaphoreType.DMA,
        ],
    )
    def edge_kernel(table_hbm, src_hbm, dst_hbm, out_hbm,
                    sidx0, didx0, sidx1, didx1, rows0, rows1, acc_sh,
                    sem0, sem1):
        c = lax.axis_index("c")
        s = lax.axis_index("s")

        def zrow(i, carry):
            for j in range(_D // 16):
                rows0[i, pl.ds(j * 16, 16)] = jnp.zeros((16,), jnp.float32)
            return carry
        lax.fori_loop(0, _K, zrow, 0)
        for blk in range(_STRIPE // _K):
            pltpu.sync_copy(rows0,
                            acc_sh.at[pl.ds(s * _STRIPE + blk * _K, _K), :])
        plsc.subcore_barrier()

        def fire(base, j, sidx, didx, rows, sem):
            off = base + j * _K
            pltpu.sync_copy(src_hbm.at[pl.ds(off, _K)], sidx)
            pltpu.sync_copy(dst_hbm.at[pl.ds(off, _K)], didx)
            pltpu.async_copy(table_hbm.at[sidx], rows, sem)

        def drain(sidx, didx, rows, sem):
            pltpu.make_async_copy(table_hbm.at[sidx], rows, sem).wait()
            pltpu.sync_copy(rows, acc_sh.at[didx], add=True)

        def run_chunks(nchunk, base):
            fire(base, 0, sidx0, didx0, rows0, sem0)

            def pair(jj, carry):
                a = 2 * jj
                fire(base, a + 1, sidx1, didx1, rows1, sem1)
                drain(sidx0, didx0, rows0, sem0)
                fire(base, a + 2, sidx0, didx0, rows0, sem0)
                drain(sidx1, didx1, rows1, sem1)
                return carry

            lax.fori_loop(0, (nchunk - 1) // 2, pair, 0)
            drain(sidx0, didx0, rows0, sem0)

        n0, n1 = _NCHUNK_C
        core0_edges = _NTEC * n0 * _K

        @pl.when(c == 0)
        def _():
            run_chunks(n0, s * n0 * _K)

        @pl.when(c == 1)
        def _():
            run_chunks(n1, core0_edges + s * n1 * _K)

        plsc.subcore_barrier()
        pltpu.sync_copy(
            acc_sh.at[pl.ds(s * _STRIPE, _STRIPE), :],
            out_hbm.at[c, pl.ds(s * _STRIPE, _STRIPE), :],
        )

    return edge_kernel


def _dinv_block(degp_ref, i):
    deg = degp_ref[0, :, 0:1] + degp_ref[1, :, 0:1] + 1.0
    dinv = lax.rsqrt(jnp.maximum(deg, 1.0))
    row = lax.broadcasted_iota(jnp.int32, (_BLK, 1), 0) + i * _BLK
    dinvm = jnp.where(row < _N, dinv, 0.0)
    return dinv, dinvm


def _t1_body(degp_ref, x_ref, w_ref, out_ref):
    i = pl.program_id(0)
    _, dinvm = _dinv_block(degp_ref, i)
    xw = jnp.dot(x_ref[...], w_ref[...], preferred_element_type=jnp.float32)
    out_ref[...] = xw * dinvm


def _t2_body(degp_ref, p_ref, ys_ref, b_ref, w_ref, out_ref):
    i = pl.program_id(0)
    dinv, dinvm = _dinv_block(degp_ref, i)
    ssum = p_ref[0] + p_ref[1] + ys_ref[...]
    h = jnp.maximum(ssum * dinv + b_ref[...], 0.0)
    out_ref[...] = jnp.dot(h, w_ref[...], preferred_element_type=jnp.float32) * dinvm


def _t3_body(degp_ref, p_ref, ys_ref, b_ref, batch_ref, wfc_ref, bfc_ref,
             out_ref, pooled_acc, cnt_acc):
    i = pl.program_id(0)

    @pl.when(i == 0)
    def _():
        pooled_acc[...] = jnp.zeros_like(pooled_acc)
        cnt_acc[...] = jnp.zeros_like(cnt_acc)

    dinv, _ = _dinv_block(degp_ref, i)
    h3 = (p_ref[0] + p_ref[1] + ys_ref[...]) * dinv + b_ref[...]
    bb = batch_ref[pl.ds(i * _BLK, _BLK)]
    onehot = (bb[None, :] == lax.broadcasted_iota(jnp.int32, (_B, _BLK), 0)
              ).astype(jnp.float32)
    pooled_acc[...] += jnp.dot(onehot, h3, preferred_element_type=jnp.float32)
    cnt_acc[...] += jnp.sum(onehot, axis=1, keepdims=True)

    @pl.when(i == _NBLK - 1)
    def _():
        pooled = pooled_acc[...] / jnp.maximum(cnt_acc[...], 1.0)
        out_ref[...] = (
            jnp.dot(pooled, wfc_ref[...], preferred_element_type=jnp.float32)
            + bfc_ref[...]
        )


def _t1(degp, x_p, W1):
    return pl.pallas_call(
        _t1_body,
        grid=(_NBLK,),
        in_specs=[
            pl.BlockSpec((_NSC, _BLK, _D), lambda i: (0, i, 0)),
            pl.BlockSpec((_BLK, _D), lambda i: (i, 0)),
            pl.BlockSpec((_D, _D), lambda i: (0, 0)),
        ],
        out_specs=pl.BlockSpec((_BLK, _D), lambda i: (i, 0)),
        out_shape=jax.ShapeDtypeStruct((_NPAD, _D), jnp.float32),
    )(degp, x_p, W1)


def _t2(degp, p, ys, b2d, Wn):
    return pl.pallas_call(
        _t2_body,
        grid=(_NBLK,),
        in_specs=[
            pl.BlockSpec((_NSC, _BLK, _D), lambda i: (0, i, 0)),
            pl.BlockSpec((_NSC, _BLK, _D), lambda i: (0, i, 0)),
            pl.BlockSpec((_BLK, _D), lambda i: (i, 0)),
            pl.BlockSpec((1, _D), lambda i: (0, 0)),
            pl.BlockSpec((_D, _D), lambda i: (0, 0)),
        ],
        out_specs=pl.BlockSpec((_BLK, _D), lambda i: (i, 0)),
        out_shape=jax.ShapeDtypeStruct((_NPAD, _D), jnp.float32),
    )(degp, p, ys, b2d, Wn)


def _t3(degp, p, ys, b2d, batch_p, Wfc, bfc2d):
    return pl.pallas_call(
        _t3_body,
        grid=(_NBLK,),
        in_specs=[
            pl.BlockSpec((_NSC, _BLK, _D), lambda i: (0, i, 0)),
            pl.BlockSpec((_NSC, _BLK, _D), lambda i: (0, i, 0)),
            pl.BlockSpec((_BLK, _D), lambda i: (i, 0)),
            pl.BlockSpec((1, _D), lambda i: (0, 0)),
            pl.BlockSpec((_NPAD,), lambda i: (0,)),
            pl.BlockSpec((_D, _C), lambda i: (0, 0)),
            pl.BlockSpec((1, _C), lambda i: (0, 0)),
        ],
        out_specs=pl.BlockSpec((_B, _C), lambda i: (0, 0)),
        out_shape=jax.ShapeDtypeStruct((_B, _C), jnp.float32),
        scratch_shapes=[
            pltpu.VMEM((_B, _D), jnp.float32),
            pltpu.VMEM((_B, 1), jnp.float32),
        ],
    )(degp, p, ys, b2d, batch_p, Wfc, bfc2d)


def kernel(x, edge_index, batch, W1, b1, W2, b2, W3, b3, Wfc, bfc):
    src = edge_index[0]
    dst = edge_index[1]
    pad_e = _EPAD - _E
    fill = jnp.full((pad_e,), _N, jnp.int32)
    src_p = jnp.concatenate([src, fill])
    dst_p = jnp.concatenate([dst, fill])
    x_p = jnp.pad(x, ((0, _NPAD - _N), (0, 0)))
    batch_p = jnp.concatenate(
        [batch, jnp.full((_NPAD - _N,), _B, jnp.int32)])

    edge_k = _build_edge_kernel()
    degp = _build_deg_kernel()(dst_p)
    ys1 = _t1(degp, x_p, W1)
    p1 = edge_k(ys1, src_p, dst_p)
    ys2 = _t2(degp, p1, ys1, b1.reshape(1, _D), W2)
    p2 = edge_k(ys2, src_p, dst_p)
    ys3 = _t2(degp, p2, ys2, b2.reshape(1, _D), W3)
    p3 = edge_k(ys3, src_p, dst_p)
    return _t3(degp, p3, ys3, b3.reshape(1, _D), batch_p, Wfc,
               bfc.reshape(1, _C))
